# Initial kernel scaffold; baseline (speedup 1.0000x reference)
import jax
import jax.numpy as jnp
from jax.experimental import pallas as pl


def kernel(pred, gold):
    raise NotImplementedError("write your pallas kernel here")


# speedup vs baseline: 2.2133x; 2.2133x over previous
import math

import jax
import jax.numpy as jnp
from jax.experimental import pallas as pl

_C = 1000
_PAD = 0
_SMOOTH = 0.1
_CONF = 1.0 - _SMOOTH
_EPS = _SMOOTH / (_C - 2)
_K = (_C - 2) * _EPS * math.log(_EPS) + _CONF * math.log(_CONF)

_ROWS_BLK = 512


def _body(g_ref, p_ref, out_ref):
    i = pl.program_id(0)

    p = p_ref[...]
    g = g_ref[...]
    mask = g != _PAD

    rowsum = jnp.sum(p, axis=1, keepdims=True)
    p0 = p[:, 0:1]
    col = jax.lax.broadcasted_iota(jnp.int32, p.shape, 1)
    pg = jnp.sum(jnp.where(col == g, p, 0.0), axis=1, keepdims=True)

    per_row = _K - (_EPS * (rowsum - p0 - pg) + _CONF * pg)
    blk = jnp.sum(jnp.where(mask, per_row, 0.0), keepdims=True).reshape(1, 1)

    @pl.when(i == 0)
    def _():
        out_ref[...] = jnp.zeros_like(out_ref)

    out_ref[...] += blk


def kernel(pred, gold):
    p = pred.reshape(-1, _C)
    n = p.shape[0]
    g = gold.reshape(n, 1).astype(jnp.int32)

    out = pl.pallas_call(
        _body,
        grid=(n // _ROWS_BLK,),
        in_specs=[
            pl.BlockSpec((_ROWS_BLK, 1), lambda i: (i, 0)),
            pl.BlockSpec((_ROWS_BLK, _C), lambda i: (i, 0)),
        ],
        out_specs=pl.BlockSpec((1, 1), lambda i: (0, 0)),
        out_shape=jax.ShapeDtypeStruct((1, 1), jnp.float32),
    )(g, p)
    return out[0, 0]


# speedup vs baseline: 2.4691x; 1.1156x over previous
import math

import jax
import jax.numpy as jnp
from jax.experimental import pallas as pl

_C = 1000
_PAD = 0
_SMOOTH = 0.1
_CONF = 1.0 - _SMOOTH
_EPS = _SMOOTH / (_C - 2)
_K = (_C - 2) * _EPS * math.log(_EPS) + _CONF * math.log(_CONF)

_ROWS_BLK = 512


def _body(g_ref, p_ref, out_ref):
    step = pl.program_id(0) * pl.num_programs(1) + pl.program_id(1)

    p = p_ref[0]
    g = g_ref[0]
    mask = g != _PAD

    rowsum = jnp.sum(p, axis=1, keepdims=True)
    p0 = p[:, 0:1]
    col = jax.lax.broadcasted_iota(jnp.int32, p.shape, 1)
    pg = jnp.sum(jnp.where(col == g, p, 0.0), axis=1, keepdims=True)

    per_row = _K - (_EPS * (rowsum - p0 - pg) + _CONF * pg)
    blk = jnp.sum(jnp.where(mask, per_row, 0.0), keepdims=True).reshape(1, 1)

    @pl.when(step == 0)
    def _():
        out_ref[...] = jnp.zeros_like(out_ref)

    out_ref[...] += blk


def kernel(pred, gold):
    b, t, c = pred.shape
    g = gold.reshape(b, t, 1).astype(jnp.int32)

    out = pl.pallas_call(
        _body,
        grid=(b, t // _ROWS_BLK),
        in_specs=[
            pl.BlockSpec((1, _ROWS_BLK, 1), lambda i, j: (i, j, 0)),
            pl.BlockSpec((1, _ROWS_BLK, c), lambda i, j: (i, j, 0)),
        ],
        out_specs=pl.BlockSpec((1, 1), lambda i, j: (0, 0)),
        out_shape=jax.ShapeDtypeStruct((1, 1), jnp.float32),
    )(g, pred)
    return out[0, 0]


# speedup vs baseline: 2.7127x; 1.0986x over previous
import math

import jax
import jax.numpy as jnp
from jax.experimental import pallas as pl

_C = 1000
_PAD = 0
_SMOOTH = 0.1
_CONF = 1.0 - _SMOOTH
_EPS = _SMOOTH / (_C - 2)
_K = (_C - 2) * _EPS * math.log(_EPS) + _CONF * math.log(_CONF)

_ROWS_BLK = 1024


def _body(g_ref, p_ref, out_ref):
    step = pl.program_id(0) * pl.num_programs(1) + pl.program_id(1)

    p = p_ref[0]
    g = g_ref[0]
    mask = g != _PAD

    rowsum = jnp.sum(p, axis=1, keepdims=True)
    p0 = p[:, 0:1]
    col = jax.lax.broadcasted_iota(jnp.int32, p.shape, 1)
    pg = jnp.sum(jnp.where(col == g, p, 0.0), axis=1, keepdims=True)

    per_row = _K - (_EPS * (rowsum - p0 - pg) + _CONF * pg)
    blk = jnp.sum(jnp.where(mask, per_row, 0.0), keepdims=True).reshape(1, 1)

    @pl.when(step == 0)
    def _():
        out_ref[...] = jnp.zeros_like(out_ref)

    out_ref[...] += blk


def kernel(pred, gold):
    b, t, c = pred.shape
    g = gold.reshape(b, t, 1).astype(jnp.int32)

    out = pl.pallas_call(
        _body,
        grid=(b, t // _ROWS_BLK),
        in_specs=[
            pl.BlockSpec((1, _ROWS_BLK, 1), lambda i, j: (i, j, 0)),
            pl.BlockSpec((1, _ROWS_BLK, c), lambda i, j: (i, j, 0)),
        ],
        out_specs=pl.BlockSpec((1, 1), lambda i, j: (0, 0)),
        out_shape=jax.ShapeDtypeStruct((1, 1), jnp.float32),
    )(g, pred)
    return out[0, 0]


# speedup vs baseline: 2.7791x; 1.0245x over previous
import math

import jax
import jax.numpy as jnp
from jax.experimental import pallas as pl

_C = 1000
_PAD = 0
_SMOOTH = 0.1
_CONF = 1.0 - _SMOOTH
_EPS = _SMOOTH / (_C - 2)
_K = (_C - 2) * _EPS * math.log(_EPS) + _CONF * math.log(_CONF)

_ROWS_BLK = 2048


def _body(g_ref, p_ref, out_ref):
    step = pl.program_id(0) * pl.num_programs(1) + pl.program_id(1)

    p = p_ref[0]
    g = g_ref[0]
    mask = g != _PAD

    rowsum = jnp.sum(p, axis=1, keepdims=True)
    p0 = p[:, 0:1]
    col = jax.lax.broadcasted_iota(jnp.int32, p.shape, 1)
    pg = jnp.sum(jnp.where(col == g, p, 0.0), axis=1, keepdims=True)

    per_row = _K - (_EPS * (rowsum - p0 - pg) + _CONF * pg)
    blk = jnp.sum(jnp.where(mask, per_row, 0.0), keepdims=True).reshape(1, 1)

    @pl.when(step == 0)
    def _():
        out_ref[...] = jnp.zeros_like(out_ref)

    out_ref[...] += blk


def kernel(pred, gold):
    b, t, c = pred.shape
    g = gold.reshape(b, t, 1).astype(jnp.int32)

    out = pl.pallas_call(
        _body,
        grid=(b, t // _ROWS_BLK),
        in_specs=[
            pl.BlockSpec((1, _ROWS_BLK, 1), lambda i, j: (i, j, 0)),
            pl.BlockSpec((1, _ROWS_BLK, c), lambda i, j: (i, j, 0)),
        ],
        out_specs=pl.BlockSpec((1, 1), lambda i, j: (0, 0)),
        out_shape=jax.ShapeDtypeStruct((1, 1), jnp.float32),
    )(g, pred)
    return out[0, 0]
